# baseline (speedup 1.0000x reference)
import functools

import jax
import jax.numpy as jnp
from jax import lax
from jax.experimental import pallas as pl
from jax.experimental.pallas import tpu as pltpu
from jax.experimental.pallas import tpu_sc as plsc

_N, _C, _H, _W = 16, 3, 512, 512
_P = _H * _W
_TOPK = int(_P * 0.05)
_NB = 4096
_LANES = 16
_SUBSTRIDE = _NB + 16
_HIST_WORDS = _SUBSTRIDE * _LANES
_CH = 16384
_HALF = _P // 2


def _dark_body(img_ref, bmat_ref, out_ref):
    x = img_ref[0]
    m = jnp.minimum(jnp.minimum(x[0], x[1]), x[2])

    s15 = jnp.dot(m, bmat_ref[...], preferred_element_type=jnp.float32)

    zv = jnp.zeros((8, _W), jnp.float32)
    u = jnp.concatenate([zv, s15, zv], axis=0)
    q1 = u[:525] + u[1:526]
    q2 = q1[:521] + q1[2:523]
    q3 = q2[:513] + q2[4:517]
    dk = (q3[1:513] + q2[9:521] + q1[13:525] + u[15:527]) * (1.0 / 225.0)
    for p in range(4):
        out_ref[0, p] = dk[:, 128 * p:128 * (p + 1)]


@functools.cache
def _hist_call(n_imgs):
    mesh = plsc.VectorSubcoreMesh(
        core_axis_name="c", subcore_axis_name="s",
        num_cores=2, num_subcores=16)
    return pl.kernel(
        functools.partial(_hist_sc_body, n_imgs * _P // 32),
        out_type=jax.ShapeDtypeStruct((32 * _NB,), jnp.int32),
        mesh=mesh,
        compiler_params=pltpu.CompilerParams(needs_layout_passes=False),
        scratch_types=[
            pltpu.VMEM((_CH,), jnp.float32),
            pltpu.VMEM((_CH,), jnp.float32),
            pltpu.VMEM((_HIST_WORDS,), jnp.int32),
            pltpu.VMEM((_NB,), jnp.int32),
            pltpu.SemaphoreType.DMA,
            pltpu.SemaphoreType.DMA,
        ],
    )


def _hist_sc_body(px_per_tile, dark_hbm, out_hbm, buf_a, buf_b, hist, merged,
                  sem_a, sem_b):
    wid = lax.axis_index("s") * 2 + lax.axis_index("c")
    base = wid * px_per_tile

    zeros16 = jnp.zeros((_LANES,), jnp.int32)

    def zbody(j):
        hist[pl.ds(j * _LANES, _LANES)] = zeros16

    plsc.parallel_loop(0, _HIST_WORDS // _LANES, unroll=8)(zbody)

    lane_off = lax.broadcasted_iota(jnp.int32, (_LANES,), 0) * (_SUBSTRIDE + 1)
    ones16 = jnp.ones((_LANES,), jnp.int32)

    bufs = (buf_a, buf_b)
    sems = (sem_a, sem_b)
    nchunk = px_per_tile // _CH
    copies = [pltpu.async_copy(dark_hbm.at[pl.ds(base, _CH)],
                               buf_a, sem_a)]
    for ci in range(nchunk):
        if ci + 1 < nchunk:
            copies.append(pltpu.async_copy(
                dark_hbm.at[pl.ds(base + (ci + 1) * _CH, _CH)],
                bufs[(ci + 1) % 2], sems[(ci + 1) % 2]))
        copies[ci].wait()
        buf = bufs[ci % 2]

        def gbody(j, buf=buf):
            v = buf[pl.ds(j * _LANES, _LANES)]
            b = jnp.minimum((v * float(_NB)).astype(jnp.int32), _NB - 1)
            plsc.addupdate_scatter(hist, [b + lane_off], ones16)

        plsc.parallel_loop(0, _CH // _LANES, unroll=8)(gbody)

    def mbody(g):
        acc = jnp.zeros((_LANES,), jnp.int32)
        for lane in range(_LANES):
            acc = acc + hist[pl.ds(g * _LANES + lane * (_SUBSTRIDE + 1),
                                   _LANES)]
        merged[pl.ds(g * _LANES, _LANES)] = acc

    plsc.parallel_loop(0, _NB // _LANES, unroll=2)(mbody)

    pltpu.sync_copy(merged, out_hbm.at[pl.ds(wid * _NB, _NB)])


def _fused_body(w_ref, parts_ref, img_ref, dark_ref, out_ref):
    i = pl.program_id(0)
    x = parts_ref[0]
    h = (x[0] + x[1]) + (x[2] + x[3])

    s = h
    sh = 1
    while sh < _NB:
        s = s + jnp.concatenate([s[sh:], jnp.zeros((sh,), jnp.int32)])
        sh *= 2
    c_above_bins = s - h
    crit = jnp.sum((c_above_bins >= _TOPK).astype(jnp.int32))

    bins = lax.broadcasted_iota(jnp.int32, (_NB,), 0)
    c_above = jnp.sum(jnp.where(bins > crit, h, 0)).astype(jnp.float32)
    c_bin = jnp.sum(jnp.where(bins == crit, h, 0)).astype(jnp.float32)
    frac = (float(_TOPK) - c_above) / jnp.maximum(c_bin, 1.0)
    frac = jnp.clip(frac, 0.0, 1.0)

    img0 = img_ref[0]
    w = w_ref[i]
    wgt, rcp = [], []
    for p in range(4):
        d = dark_ref[0, p]
        b = jnp.minimum((d * float(_NB)).astype(jnp.int32), _NB - 1)
        wgt.append(jnp.where(b > crit, 1.0,
                             jnp.where(b == crit, frac, 0.0)))
        rcp.append(1.0 / (jnp.maximum(1.0 - w * d, 0.1) + 0.001))
    for c in range(3):
        atm = sum(
            jnp.sum(wgt[p] * img0[c, :, 128 * p:128 * (p + 1)])
            for p in range(4)) * (1.0 / _TOPK)
        for p in range(4):
            ip = img0[c, :, 128 * p:128 * (p + 1)]
            out_ref[0, c, :, 128 * p:128 * (p + 1)] = jnp.clip(
                (ip - atm) * rcp[p] + atm, 0.0, 1.0)


def _dark_call(img, start, n):
    cols = lax.broadcasted_iota(jnp.int32, (_W, _W), 1)
    rows = lax.broadcasted_iota(jnp.int32, (_W, _W), 0)
    bmat = (jnp.abs(cols - rows) <= 7).astype(jnp.float32)
    return pl.pallas_call(
        _dark_body,
        grid=(n,),
        in_specs=[
            pl.BlockSpec((1, _C, _H, _W), lambda i: (i + start, 0, 0, 0)),
            pl.BlockSpec((_W, _W), lambda i: (0, 0)),
        ],
        out_specs=pl.BlockSpec((1, 4, _H, 128), lambda i: (i, 0, 0, 0)),
        out_shape=jax.ShapeDtypeStruct((n, 4, _H, 128), jnp.float32),
    )(img, bmat)


def _fused_call(w, parts, img, dark):
    return pl.pallas_call(
        _fused_body,
        grid=(_N,),
        in_specs=[
            pl.BlockSpec(memory_space=pltpu.SMEM),
            pl.BlockSpec((1, 4, _NB), lambda i: (i, 0, 0)),
            pl.BlockSpec((1, _C, _H, _W), lambda i: (i, 0, 0, 0)),
            pl.BlockSpec((1, 4, _H, 128), lambda i: (i, 0, 0, 0)),
        ],
        out_specs=pl.BlockSpec((1, _C, _H, _W), lambda i: (i, 0, 0, 0)),
        out_shape=jax.ShapeDtypeStruct((_N, _C, _H, _W), jnp.float32),
    )(w, parts, img, dark)


def kernel(img, w):
    half = _N // 2
    dark_a = _dark_call(img, 0, half)
    parts_a = _hist_call(half)(dark_a.reshape(half * _P))
    dark_b = _dark_call(img, half, half)
    parts_b = _hist_call(half)(dark_b.reshape(half * _P))
    dark = jnp.concatenate([dark_a, dark_b], axis=0)
    parts = jnp.concatenate([parts_a, parts_b]).reshape(_N, 4, _NB)
    return _fused_call(w, parts, img, dark)


# speedup vs baseline: 1.0443x; 1.0443x over previous
import functools

import jax
import jax.numpy as jnp
from jax import lax
from jax.experimental import pallas as pl
from jax.experimental.pallas import tpu as pltpu
from jax.experimental.pallas import tpu_sc as plsc

_N, _C, _H, _W = 16, 3, 512, 512
_P = _H * _W
_TOPK = int(_P * 0.05)
_NB = 4096
_LANES = 16
_SUBSTRIDE = _NB + 16
_HIST_WORDS = _SUBSTRIDE * _LANES
_CH = 16384
_HALF = _P // 2


def _dark_body(img_ref, bmat_ref, out_ref):
    x = img_ref[0]
    m = jnp.minimum(jnp.minimum(x[0], x[1]), x[2])

    s15 = jnp.dot(m, bmat_ref[...], preferred_element_type=jnp.float32)

    zv = jnp.zeros((8, _W), jnp.float32)
    u = jnp.concatenate([zv, s15, zv], axis=0)
    q1 = u[:525] + u[1:526]
    q2 = q1[:521] + q1[2:523]
    q3 = q2[:513] + q2[4:517]
    dk = (q3[1:513] + q2[9:521] + q1[13:525] + u[15:527]) * (1.0 / 225.0)
    for p in range(4):
        out_ref[0, p] = dk[:, 128 * p:128 * (p + 1)]


@functools.cache
def _hist_call(n_imgs):
    mesh = plsc.VectorSubcoreMesh(
        core_axis_name="c", subcore_axis_name="s",
        num_cores=2, num_subcores=16)
    return pl.kernel(
        functools.partial(_hist_sc_body, n_imgs * _P // 32),
        out_type=jax.ShapeDtypeStruct((32 * _NB,), jnp.int32),
        mesh=mesh,
        compiler_params=pltpu.CompilerParams(needs_layout_passes=False),
        scratch_types=[
            pltpu.VMEM((_CH,), jnp.float32),
            pltpu.VMEM((_CH,), jnp.float32),
            pltpu.VMEM((_HIST_WORDS,), jnp.int32),
            pltpu.VMEM((_NB,), jnp.int32),
            pltpu.SemaphoreType.DMA,
            pltpu.SemaphoreType.DMA,
        ],
    )


def _hist_sc_body(px_per_tile, dark_hbm, out_hbm, buf_a, buf_b, hist, merged,
                  sem_a, sem_b):
    wid = lax.axis_index("s") * 2 + lax.axis_index("c")
    base = wid * px_per_tile

    zeros16 = jnp.zeros((_LANES,), jnp.int32)

    def zbody(j):
        hist[pl.ds(j * _LANES, _LANES)] = zeros16

    plsc.parallel_loop(0, _HIST_WORDS // _LANES, unroll=8)(zbody)

    lane_off = lax.broadcasted_iota(jnp.int32, (_LANES,), 0) * (_SUBSTRIDE + 1)
    ones16 = jnp.ones((_LANES,), jnp.int32)

    bufs = (buf_a, buf_b)
    sems = (sem_a, sem_b)
    nchunk = px_per_tile // _CH
    copies = [pltpu.async_copy(dark_hbm.at[pl.ds(base, _CH)],
                               buf_a, sem_a)]
    for ci in range(nchunk):
        if ci + 1 < nchunk:
            copies.append(pltpu.async_copy(
                dark_hbm.at[pl.ds(base + (ci + 1) * _CH, _CH)],
                bufs[(ci + 1) % 2], sems[(ci + 1) % 2]))
        copies[ci].wait()
        buf = bufs[ci % 2]

        def gbody(j, buf=buf):
            v = buf[pl.ds(j * _LANES, _LANES)]
            b = jnp.minimum((v * float(_NB)).astype(jnp.int32), _NB - 1)
            plsc.addupdate_scatter(hist, [b + lane_off], ones16)

        plsc.parallel_loop(0, _CH // _LANES, unroll=8)(gbody)

    def mbody(g):
        acc = jnp.zeros((_LANES,), jnp.int32)
        for lane in range(_LANES):
            acc = acc + hist[pl.ds(g * _LANES + lane * (_SUBSTRIDE + 1),
                                   _LANES)]
        merged[pl.ds(g * _LANES, _LANES)] = acc

    plsc.parallel_loop(0, _NB // _LANES, unroll=2)(mbody)

    pltpu.sync_copy(merged, out_hbm.at[pl.ds(wid * _NB, _NB)])


def _fused_body(w_ref, parts_ref, img_ref, dark_ref, out_ref):
    i = pl.program_id(0)
    x = parts_ref[0]
    h = x[0] + x[1]

    s = h
    sh = 1
    while sh < _NB:
        s = s + jnp.concatenate([s[sh:], jnp.zeros((sh,), jnp.int32)])
        sh *= 2
    c_above_bins = s - h
    crit = jnp.sum((c_above_bins >= _TOPK).astype(jnp.int32))

    bins = lax.broadcasted_iota(jnp.int32, (_NB,), 0)
    c_above = jnp.sum(jnp.where(bins > crit, h, 0)).astype(jnp.float32)
    c_bin = jnp.sum(jnp.where(bins == crit, h, 0)).astype(jnp.float32)
    frac = (float(_TOPK) - c_above) / jnp.maximum(c_bin, 1.0)
    frac = jnp.clip(frac, 0.0, 1.0)

    img0 = img_ref[0]
    w = w_ref[i]
    wgt, rcp = [], []
    for p in range(4):
        d = dark_ref[0, p]
        b = jnp.minimum((d * float(_NB)).astype(jnp.int32), _NB - 1)
        wgt.append(jnp.where(b > crit, 1.0,
                             jnp.where(b == crit, frac, 0.0)))
        rcp.append(1.0 / (jnp.maximum(1.0 - w * d, 0.1) + 0.001))
    for c in range(3):
        atm = sum(
            jnp.sum(wgt[p] * img0[c, :, 128 * p:128 * (p + 1)])
            for p in range(4)) * (1.0 / _TOPK)
        for p in range(4):
            ip = img0[c, :, 128 * p:128 * (p + 1)]
            out_ref[0, c, :, 128 * p:128 * (p + 1)] = jnp.clip(
                (ip - atm) * rcp[p] + atm, 0.0, 1.0)


def _dark_call(img, start, n):
    cols = lax.broadcasted_iota(jnp.int32, (_W, _W), 1)
    rows = lax.broadcasted_iota(jnp.int32, (_W, _W), 0)
    bmat = (jnp.abs(cols - rows) <= 7).astype(jnp.float32)
    return pl.pallas_call(
        _dark_body,
        grid=(n,),
        in_specs=[
            pl.BlockSpec((1, _C, _H, _W), lambda i: (i + start, 0, 0, 0)),
            pl.BlockSpec((_W, _W), lambda i: (0, 0)),
        ],
        out_specs=pl.BlockSpec((1, 4, _H, 128), lambda i: (i, 0, 0, 0)),
        out_shape=jax.ShapeDtypeStruct((n, 4, _H, 128), jnp.float32),
    )(img, bmat)


def _fused_call(w, parts, img, dark):
    return pl.pallas_call(
        _fused_body,
        grid=(_N,),
        in_specs=[
            pl.BlockSpec(memory_space=pltpu.SMEM),
            pl.BlockSpec((1, 2, _NB), lambda i: (i, 0, 0)),
            pl.BlockSpec((1, _C, _H, _W), lambda i: (i, 0, 0, 0)),
            pl.BlockSpec((1, 4, _H, 128), lambda i: (i, 0, 0, 0)),
        ],
        out_specs=pl.BlockSpec((1, _C, _H, _W), lambda i: (i, 0, 0, 0)),
        out_shape=jax.ShapeDtypeStruct((_N, _C, _H, _W), jnp.float32),
    )(w, parts, img, dark)


def kernel(img, w):
    dark = _dark_call(img, 0, _N)
    parts = _hist_call(_N)(dark.reshape(_N * _P))
    return _fused_call(w, parts.reshape(_N, 2, _NB), img, dark)
